# baseline (speedup 1.0000x reference)
import functools

import jax
import jax.numpy as jnp
from jax import lax
from jax.experimental import pallas as pl
from jax.experimental.pallas import tpu as pltpu
from jax.experimental.pallas import tpu_sc as plsc

D = 1024
C = 32
NC = 2
NS = 16
NW = NC * NS
L = 16
UNROLL = 16


def _sc_body(x_hbm, pos_hbm, out_hbm, pos_v, x_v):
    bsd = x_hbm.shape[0]
    sd = pos_hbm.shape[0]
    nbatch = bsd // sd
    seq_per_w = sd // D // NW

    wid = lax.axis_index("s") * NC + lax.axis_index("c")
    w0 = wid * seq_per_w * D

    def add_block(i, carry):
        base = i * (UNROLL * L)
        for u in range(UNROLL):
            off = base + u * L
            plsc.addupdate(x_v.at[pl.ds(off, L)], pos_v[pl.ds(off, L)])
        return carry

    for it in range(seq_per_w // C):
        e0 = w0 + it * C * D
        pltpu.sync_copy(pos_hbm.at[pl.ds(e0, C * D)], pos_v)
        for b in range(nbatch):
            pltpu.sync_copy(x_hbm.at[pl.ds(b * sd + e0, C * D)], x_v)
            lax.fori_loop(0, C * D // (UNROLL * L), add_block, None)
            pltpu.sync_copy(x_v, out_hbm.at[pl.ds(b * sd + e0, C * D)])


def kernel(x, pos_table):
    b, s, d = x.shape
    xf = x.reshape(b * s * d)
    pos = pos_table[:s].reshape(s * d)

    mesh = plsc.VectorSubcoreMesh(core_axis_name="c", subcore_axis_name="s")
    run = functools.partial(
        pl.kernel,
        mesh=mesh,
        out_type=jax.ShapeDtypeStruct((b * s * d,), jnp.float32),
        scratch_types=[
            pltpu.VMEM((C * D,), jnp.float32),
            pltpu.VMEM((C * D,), jnp.float32),
        ],
    )(_sc_body)
    out = run(xf, pos)
    return out.reshape(b, s, d)


# speedup vs baseline: 1.2195x; 1.2195x over previous
import functools

import jax
import jax.numpy as jnp
from jax import lax
from jax.experimental import pallas as pl
from jax.experimental.pallas import tpu as pltpu
from jax.experimental.pallas import tpu_sc as plsc

D = 1024
C = 16
NC = 2
NS = 16
NW = NC * NS
L = 16
UNROLL = 16
NXB = 4
NPB = 2


def _sc_body(x_hbm, pos_hbm, out_hbm, *scratch):
    xv = scratch[0:NXB]
    pv = scratch[NXB:NXB + NPB]
    xs = scratch[NXB + NPB:NXB + NPB + NXB]
    os_ = scratch[NXB + NPB + NXB:NXB + NPB + 2 * NXB]
    ps = scratch[NXB + NPB + 2 * NXB:]

    bsd = x_hbm.shape[0]
    sd = pos_hbm.shape[0]
    nbatch = bsd // sd
    seq_per_w = sd // D // NW
    nchunks = seq_per_w // C
    nitems = nchunks * nbatch

    wid = lax.axis_index("s") * NC + lax.axis_index("c")
    w0 = wid * seq_per_w * D

    def chunk_off(it):
        return w0 + it * C * D

    def start_xload(k):
        it, b = divmod(k, nbatch)
        j = k % NXB
        return pltpu.async_copy(
            x_hbm.at[pl.ds(b * sd + chunk_off(it), C * D)], xv[j], xs[j])

    def start_posload(it):
        j = it % NPB
        return pltpu.async_copy(
            pos_hbm.at[pl.ds(chunk_off(it), C * D)], pv[j], ps[j])

    pos_loads = {0: start_posload(0)}
    x_loads = {k: start_xload(k) for k in range(min(NXB - 1, nitems))}
    last_store = [None] * NXB

    for k in range(nitems):
        it, b = divmod(k, nbatch)
        j = k % NXB
        if b == 0:
            pos_loads.pop(it).wait()
            if it + 1 < nchunks:
                pos_loads[it + 1] = start_posload(it + 1)
        x_loads.pop(k).wait()

        xbuf = xv[j]
        pbuf = pv[it % NPB]

        @plsc.parallel_loop(0, C * D, step=L, unroll=UNROLL)
        def _(i):
            plsc.addupdate(xbuf.at[pl.ds(i, L)], pbuf[pl.ds(i, L)])

        last_store[j] = pltpu.async_copy(
            xbuf, out_hbm.at[pl.ds(b * sd + chunk_off(it), C * D)], os_[j])

        n = k + NXB - 1
        if n < nitems:
            jn = n % NXB
            if last_store[jn] is not None:
                last_store[jn].wait()
                last_store[jn] = None
            x_loads[n] = start_xload(n)

    for st in last_store:
        if st is not None:
            st.wait()


def kernel(x, pos_table):
    b, s, d = x.shape
    xf = x.reshape(b * s * d)
    pos = pos_table[:s].reshape(s * d)

    mesh = plsc.VectorSubcoreMesh(core_axis_name="c", subcore_axis_name="s")
    run = functools.partial(
        pl.kernel,
        mesh=mesh,
        out_type=jax.ShapeDtypeStruct((b * s * d,), jnp.float32),
        scratch_types=(
            [pltpu.VMEM((C * D,), jnp.float32) for _ in range(NXB)]
            + [pltpu.VMEM((C * D,), jnp.float32) for _ in range(NPB)]
            + [pltpu.SemaphoreType.DMA for _ in range(2 * NXB + NPB)]
        ),
    )(_sc_body)
    out = run(xf, pos)
    return out.reshape(b, s, d)


# speedup vs baseline: 3.7844x; 3.1032x over previous
import functools

import jax
import jax.numpy as jnp
from jax import lax
from jax.experimental import pallas as pl
from jax.experimental.pallas import tpu as pltpu
from jax.experimental.pallas import tpu_sc as plsc

D = 1024
C = 16
NC = 2
NS = 16
NW = NC * NS
L = 16
UNROLL = 16
NXB = 4
NPB = 2
CPR = D // L


def _sc_body(x_hbm, pos_hbm, out_hbm, *scratch):
    xv = scratch[0:NXB]
    pv = scratch[NXB:NXB + NPB]
    xs = scratch[NXB + NPB:NXB + NPB + NXB]
    os_ = scratch[NXB + NPB + NXB:NXB + NPB + 2 * NXB]
    ps = scratch[NXB + NPB + 2 * NXB:]

    nbatch, s, _ = x_hbm.shape
    seq_per_w = s // NW
    nchunks = seq_per_w // C
    nitems = nchunks * nbatch

    wid = lax.axis_index("s") * NC + lax.axis_index("c")
    w0 = wid * seq_per_w

    def start_xload(k):
        it, b = divmod(k, nbatch)
        j = k % NXB
        return pltpu.async_copy(
            x_hbm.at[b, pl.ds(w0 + it * C, C)], xv[j], xs[j])

    def start_posload(it):
        j = it % NPB
        return pltpu.async_copy(
            pos_hbm.at[pl.ds(w0 + it * C, C)], pv[j], ps[j])

    pos_loads = {0: start_posload(0)}
    x_loads = {k: start_xload(k) for k in range(min(NXB - 1, nitems))}
    last_store = [None] * NXB

    for k in range(nitems):
        it, b = divmod(k, nbatch)
        j = k % NXB
        if b == 0:
            pos_loads.pop(it).wait()
            if it + 1 < nchunks:
                pos_loads[it + 1] = start_posload(it + 1)
        x_loads.pop(k).wait()

        xbuf = xv[j]
        pbuf = pv[it % NPB]

        @plsc.parallel_loop(0, C * CPR, step=1, unroll=UNROLL)
        def _(n):
            r = lax.shift_right_logical(n, 6)
            c = pl.multiple_of(lax.shift_left(lax.bitwise_and(n, CPR - 1), 4), L)
            plsc.addupdate(xbuf.at[r, pl.ds(c, L)], pbuf[r, pl.ds(c, L)])

        last_store[j] = pltpu.async_copy(
            xbuf, out_hbm.at[b, pl.ds(w0 + it * C, C)], os_[j])

        n = k + NXB - 1
        if n < nitems:
            jn = n % NXB
            if last_store[jn] is not None:
                last_store[jn].wait()
                last_store[jn] = None
            x_loads[n] = start_xload(n)

    for st in last_store:
        if st is not None:
            st.wait()


def kernel(x, pos_table):
    b, s, d = x.shape

    mesh = plsc.VectorSubcoreMesh(core_axis_name="c", subcore_axis_name="s")
    run = functools.partial(
        pl.kernel,
        mesh=mesh,
        out_type=jax.ShapeDtypeStruct((b, s, d), jnp.float32),
        scratch_types=(
            [pltpu.VMEM((C, D), jnp.float32) for _ in range(NXB)]
            + [pltpu.VMEM((C, D), jnp.float32) for _ in range(NPB)]
            + [pltpu.SemaphoreType.DMA for _ in range(2 * NXB + NPB)]
        ),
    )(_sc_body)
    return run(x, pos_table)


# speedup vs baseline: 3.7988x; 1.0038x over previous
import functools

import jax
import jax.numpy as jnp
from jax import lax
from jax.experimental import pallas as pl
from jax.experimental.pallas import tpu as pltpu
from jax.experimental.pallas import tpu_sc as plsc

D = 1024
C = 16
NC = 2
NS = 16
NW = NC * NS
L = 16
UNROLL = 16
NXB = 5
NPB = 2
CPR = D // L


def _sc_body(x_hbm, pos_hbm, out_hbm, *scratch):
    xv = scratch[0:NXB]
    pv = scratch[NXB:NXB + NPB]
    xs = scratch[NXB + NPB:NXB + NPB + NXB]
    os_ = scratch[NXB + NPB + NXB:NXB + NPB + 2 * NXB]
    ps = scratch[NXB + NPB + 2 * NXB:]

    nbatch, s, _ = x_hbm.shape
    seq_per_w = s // NW
    nchunks = seq_per_w // C
    nitems = nchunks * nbatch

    wid = lax.axis_index("s") * NC + lax.axis_index("c")
    w0 = wid * seq_per_w

    def start_xload(k):
        it, b = divmod(k, nbatch)
        j = k % NXB
        return pltpu.async_copy(
            x_hbm.at[b, pl.ds(w0 + it * C, C)], xv[j], xs[j])

    def start_posload(it):
        j = it % NPB
        return pltpu.async_copy(
            pos_hbm.at[pl.ds(w0 + it * C, C)], pv[j], ps[j])

    pos_loads = {0: start_posload(0)}
    x_loads = {k: start_xload(k) for k in range(min(NXB - 1, nitems))}
    last_store = [None] * NXB

    for k in range(nitems):
        it, b = divmod(k, nbatch)
        j = k % NXB
        if b == 0:
            pos_loads.pop(it).wait()
            if it + 1 < nchunks:
                pos_loads[it + 1] = start_posload(it + 1)
        x_loads.pop(k).wait()

        xbuf = xv[j]
        pbuf = pv[it % NPB]

        @plsc.parallel_loop(0, C * CPR, step=1, unroll=UNROLL)
        def _(n):
            r = lax.shift_right_logical(n, 6)
            c = pl.multiple_of(lax.shift_left(lax.bitwise_and(n, CPR - 1), 4), L)
            plsc.addupdate(xbuf.at[r, pl.ds(c, L)], pbuf[r, pl.ds(c, L)])

        last_store[j] = pltpu.async_copy(
            xbuf, out_hbm.at[b, pl.ds(w0 + it * C, C)], os_[j])

        n = k + NXB - 1
        if n < nitems:
            jn = n % NXB
            if last_store[jn] is not None:
                last_store[jn].wait()
                last_store[jn] = None
            x_loads[n] = start_xload(n)

    for st in last_store:
        if st is not None:
            st.wait()


def kernel(x, pos_table):
    b, s, d = x.shape

    mesh = plsc.VectorSubcoreMesh(core_axis_name="c", subcore_axis_name="s")
    run = functools.partial(
        pl.kernel,
        mesh=mesh,
        out_type=jax.ShapeDtypeStruct((b, s, d), jnp.float32),
        scratch_types=(
            [pltpu.VMEM((C, D), jnp.float32) for _ in range(NXB)]
            + [pltpu.VMEM((C, D), jnp.float32) for _ in range(NPB)]
            + [pltpu.SemaphoreType.DMA for _ in range(2 * NXB + NPB)]
        ),
    )(_sc_body)
    return run(x, pos_table)
